# Initial kernel scaffold; baseline (speedup 1.0000x reference)
import jax
import jax.numpy as jnp
from jax.experimental import pallas as pl


def kernel(stage_labels, base_pe, residual_w, scale):
    raise NotImplementedError("write your pallas kernel here")


# speedup vs baseline: 9.3344x; 9.3344x over previous
import functools

import jax
import jax.numpy as jnp
from jax import lax
from jax.experimental import pallas as pl
from jax.experimental.pallas import tpu as pltpu
from jax.experimental.pallas import tpu_sc as plsc

_MAX_STAGES = 100000
_FEAT = 64
_BATCH = 16384
_HIST = 50
_B_TOTAL = _BATCH * _HIST

_FUSE_ROWS = 2000


def _fuse_body(scale_ref, base_ref, resid_ref, out_ref):
    out_ref[...] = base_ref[...] * scale_ref[0, 0] + resid_ref[...]


def _fuse_tables(scale, base_pe, residual_w):
    grid = _MAX_STAGES // _FUSE_ROWS
    return pl.pallas_call(
        _fuse_body,
        grid=(grid,),
        in_specs=[
            pl.BlockSpec(memory_space=pltpu.SMEM),
            pl.BlockSpec((_FUSE_ROWS, _FEAT), lambda i: (i, 0)),
            pl.BlockSpec((_FUSE_ROWS, _FEAT), lambda i: (i, 0)),
        ],
        out_specs=pl.BlockSpec((_FUSE_ROWS, _FEAT), lambda i: (i, 0)),
        out_shape=jax.ShapeDtypeStruct((_MAX_STAGES, _FEAT), jnp.float32),
    )(scale.reshape(1, 1), base_pe, residual_w)


_info = plsc.get_sparse_core_info()
_NC, _NS = _info.num_cores, _info.num_subcores
_NW = _NC * _NS
_BPW = _B_TOTAL // _NW
_CH = 512
_NCHUNK = _BPW // _CH

_sc_mesh = plsc.VectorSubcoreMesh(core_axis_name="c", subcore_axis_name="s")


@functools.partial(
    pl.kernel,
    mesh=_sc_mesh,
    out_type=jax.ShapeDtypeStruct((_B_TOTAL, _FEAT), jnp.float32),
    scratch_types=[
        pltpu.VMEM((_CH,), jnp.int32),
        pltpu.VMEM((_CH, _FEAT), jnp.float32),
        pltpu.SemaphoreType.DMA,
    ],
    compiler_params=pltpu.CompilerParams(use_tc_tiling_on_sc=False),
)
def _sc_gather(idx_hbm, tab_hbm, out_hbm, idx_v, rows_v, sem):
    wid = lax.axis_index("s") * _NC + lax.axis_index("c")
    base = wid * _BPW

    def body(i, carry):
        off = pl.multiple_of(base + i * _CH, _CH)
        pltpu.sync_copy(idx_hbm.at[pl.ds(off, _CH)], idx_v)
        pltpu.async_copy(tab_hbm.at[idx_v], rows_v, sem).wait()
        pltpu.sync_copy(rows_v, out_hbm.at[pl.ds(off, _CH)])
        return carry

    lax.fori_loop(0, _NCHUNK, body, 0)


def kernel(stage_labels, base_pe, residual_w, scale):
    combined = _fuse_tables(scale, base_pe, residual_w)
    idx = stage_labels.reshape(_B_TOTAL).astype(jnp.int32)
    out = _sc_gather(idx, combined)
    return out.reshape(_BATCH, _HIST, _FEAT)


# speedup vs baseline: 9.9446x; 1.0654x over previous
import functools

import jax
import jax.numpy as jnp
from jax import lax
from jax.experimental import pallas as pl
from jax.experimental.pallas import tpu as pltpu
from jax.experimental.pallas import tpu_sc as plsc

_MAX_STAGES = 100000
_FEAT = 64
_BATCH = 16384
_HIST = 50
_B_TOTAL = _BATCH * _HIST

_FUSE_ROWS = 2000


def _fuse_body(scale_ref, base_ref, resid_ref, out_ref):
    out_ref[...] = base_ref[...] * scale_ref[0, 0] + resid_ref[...]


def _fuse_tables(scale, base_pe, residual_w):
    grid = _MAX_STAGES // _FUSE_ROWS
    return pl.pallas_call(
        _fuse_body,
        grid=(grid,),
        in_specs=[
            pl.BlockSpec(memory_space=pltpu.SMEM),
            pl.BlockSpec((_FUSE_ROWS, _FEAT), lambda i: (i, 0)),
            pl.BlockSpec((_FUSE_ROWS, _FEAT), lambda i: (i, 0)),
        ],
        out_specs=pl.BlockSpec((_FUSE_ROWS, _FEAT), lambda i: (i, 0)),
        out_shape=jax.ShapeDtypeStruct((_MAX_STAGES, _FEAT), jnp.float32),
    )(scale.reshape(1, 1), base_pe, residual_w)


_info = plsc.get_sparse_core_info()
_NC, _NS = _info.num_cores, _info.num_subcores
_NW = _NC * _NS
_BPW = _B_TOTAL // _NW
_CH = 512
_NCHUNK = _BPW // _CH

_NPAIR = _NCHUNK // 2

_sc_mesh = plsc.VectorSubcoreMesh(core_axis_name="c", subcore_axis_name="s")


@functools.partial(
    pl.kernel,
    mesh=_sc_mesh,
    out_type=jax.ShapeDtypeStruct((_B_TOTAL, _FEAT), jnp.float32),
    scratch_types=[
        pltpu.VMEM((_NCHUNK, _CH), jnp.int32),
        pltpu.VMEM((_CH, _FEAT), jnp.float32),
        pltpu.VMEM((_CH, _FEAT), jnp.float32),
        pltpu.SemaphoreType.DMA,
        pltpu.SemaphoreType.DMA,
    ],
    compiler_params=pltpu.CompilerParams(use_tc_tiling_on_sc=False),
)
def _sc_gather(idx_hbm, tab_hbm, out_hbm, idx_v, buf0, buf1, sem0, sem1):
    wid = lax.axis_index("s") * _NC + lax.axis_index("c")
    base = wid * _BPW

    pltpu.sync_copy(idx_hbm.at[wid], idx_v)

    def g_start(i, buf, sem):
        pltpu.async_copy(tab_hbm.at[idx_v.at[i]], buf, sem)

    def g_wait(i, buf, sem):
        pltpu.make_async_copy(tab_hbm.at[idx_v.at[i]], buf, sem).wait()

    def store(i, buf):
        off = pl.multiple_of(base + i * _CH, _CH)
        pltpu.sync_copy(buf, out_hbm.at[pl.ds(off, _CH)])

    g_start(0, buf0, sem0)
    g_start(1, buf1, sem1)

    def body(j, carry):
        a = 2 * j
        g_wait(a, buf0, sem0)
        store(a, buf0)
        g_start(a + 2, buf0, sem0)
        g_wait(a + 1, buf1, sem1)
        store(a + 1, buf1)
        g_start(a + 3, buf1, sem1)
        return carry

    lax.fori_loop(0, _NPAIR - 1, body, 0)

    a = 2 * (_NPAIR - 1)
    g_wait(a, buf0, sem0)
    store(a, buf0)
    g_wait(a + 1, buf1, sem1)
    store(a + 1, buf1)


def kernel(stage_labels, base_pe, residual_w, scale):
    combined = _fuse_tables(scale, base_pe, residual_w)
    idx = stage_labels.reshape(_NW, _NCHUNK, _CH).astype(jnp.int32)
    out = _sc_gather(idx, combined)
    return out.reshape(_BATCH, _HIST, _FEAT)
